# baseline (speedup 1.0000x reference)
import functools

import jax
import jax.numpy as jnp
from jax import lax
from jax.experimental import pallas as pl
from jax.experimental.pallas import tpu as pltpu
from jax.experimental.pallas import tpu_sc as plsc

N = 10000
E = 320000
F_IN = 128
HID = 200
WS = 128
WB = HID - WS
OUT_LL = 100
CLL = 128

NC = 2
NS = 16
NW = NC * NS
EPW = E // NW
EBLK = 80
WBLK = EPW // EBLK
RPT = 624
TAIL = N - RPT * NS
DEGW = 8

_mesh = plsc.VectorSubcoreMesh(core_axis_name="c", subcore_axis_name="s")
_sc_params = pltpu.CompilerParams(use_tc_tiling_on_sc=False)


def _init_rows(src_hbm, acc, sid):
    r0 = sid * RPT
    pltpu.sync_copy(src_hbm.at[pl.ds(r0, RPT)], acc.at[pl.ds(r0, RPT)])

    @pl.when(sid == NS - 1)
    def _():
        pltpu.sync_copy(src_hbm.at[pl.ds(N - TAIL, TAIL)],
                        acc.at[pl.ds(N - TAIL, TAIL)])


def _writeback_rows(acc, out_hbm, cid, sid):
    r0 = sid * RPT
    pltpu.sync_copy(acc.at[pl.ds(r0, RPT)], out_hbm.at[cid, pl.ds(r0, RPT)])

    @pl.when(sid == NS - 1)
    def _():
        pltpu.sync_copy(acc.at[pl.ds(N - TAIL, TAIL)],
                        out_hbm.at[cid, pl.ds(N - TAIL, TAIL)])


def _pipe(s_hbm, acc, sidx, didx, rows0, rows1, g0, g1, t0, t1, nblk):
    pltpu.async_copy(s_hbm.at[sidx.at[0]], rows0, g0)

    def body(k, carry):
        @pl.when(k % 2 == 0)
        def _():
            @pl.when(k + 1 < nblk)
            def _():
                @pl.when(k >= 1)
                def _():
                    pltpu.make_async_copy(rows1, acc.at[didx.at[0]],
                                          t1).wait()

                pltpu.async_copy(s_hbm.at[sidx.at[k + 1]], rows1, g1)

            pltpu.make_async_copy(s_hbm.at[sidx.at[0]], rows0, g0).wait()
            pltpu.async_copy(rows0, acc.at[didx.at[k]], t0, add=True)

        @pl.when(k % 2 == 1)
        def _():
            @pl.when(k + 1 < nblk)
            def _():
                pltpu.make_async_copy(rows0, acc.at[didx.at[0]], t0).wait()
                pltpu.async_copy(s_hbm.at[sidx.at[k + 1]], rows0, g0)

            pltpu.make_async_copy(s_hbm.at[sidx.at[1]], rows1, g1).wait()
            pltpu.async_copy(rows1, acc.at[didx.at[k]], t1, add=True)

        return carry

    lax.fori_loop(0, nblk, body, 0)
    pltpu.make_async_copy(rows1, acc.at[didx.at[0]], t1).wait()
    pltpu.make_async_copy(rows0, acc.at[didx.at[0]], t0).wait()


@functools.partial(
    pl.kernel,
    out_type=jax.ShapeDtypeStruct((NC, N, DEGW), jnp.float32),
    mesh=_mesh,
    compiler_params=_sc_params,
    scratch_types=[
        pltpu.VMEM((WBLK, EBLK), jnp.int32),
        pltpu.VMEM((EBLK, DEGW), jnp.float32),
        pltpu.VMEM_SHARED((N, DEGW), jnp.float32),
    ],
)
def _sc_degree(dst_hbm, zeros_hbm, ones_hbm, out_hbm, didx, ones_v, acc):
    cid = lax.axis_index("c")
    sid = lax.axis_index("s")
    wid = sid * NC + cid
    _init_rows(zeros_hbm, acc, sid)
    pltpu.sync_copy(ones_hbm, ones_v)
    pltpu.sync_copy(dst_hbm.at[wid], didx)
    plsc.subcore_barrier()

    def body(j, carry):
        pltpu.sync_copy(ones_v, acc.at[didx.at[j]], add=True)
        return carry

    lax.fori_loop(0, WBLK, body, 0)
    plsc.subcore_barrier()
    _writeback_rows(acc, out_hbm, cid, sid)


@functools.partial(
    pl.kernel,
    out_type=jax.ShapeDtypeStruct((NC, N, F_IN), jnp.float32),
    mesh=_mesh,
    compiler_params=_sc_params,
    scratch_types=[
        pltpu.VMEM((WBLK, EBLK), jnp.int32),
        pltpu.VMEM((WBLK, EBLK), jnp.int32),
        pltpu.VMEM((EBLK, F_IN), jnp.float32),
        pltpu.VMEM((EBLK, F_IN), jnp.float32),
        pltpu.SemaphoreType.DMA,
        pltpu.SemaphoreType.DMA,
        pltpu.SemaphoreType.DMA,
        pltpu.SemaphoreType.DMA,
        pltpu.VMEM_SHARED((N, F_IN), jnp.float32),
    ],
)
def _sc_agg_x(u_hbm, zeros_hbm, src_hbm, dst_hbm, out_hbm, sidx, didx,
              rows0, rows1, g0, g1, t0, t1, acc):
    cid = lax.axis_index("c")
    sid = lax.axis_index("s")
    wid = sid * NC + cid

    @pl.when(cid == 0)
    def _():
        _init_rows(u_hbm, acc, sid)

    @pl.when(cid != 0)
    def _():
        _init_rows(zeros_hbm, acc, sid)

    pltpu.sync_copy(src_hbm.at[wid], sidx)
    pltpu.sync_copy(dst_hbm.at[wid], didx)
    plsc.subcore_barrier()
    _pipe(u_hbm, acc, sidx, didx, rows0, rows1, g0, g1, t0, t1, WBLK)
    plsc.subcore_barrier()
    _writeback_rows(acc, out_hbm, cid, sid)


@functools.partial(
    pl.kernel,
    out_type=jax.ShapeDtypeStruct((NC, N, WS), jnp.float32),
    mesh=_mesh,
    compiler_params=_sc_params,
    scratch_types=[
        pltpu.VMEM((WBLK, EBLK), jnp.int32),
        pltpu.VMEM((WBLK, EBLK), jnp.int32),
        pltpu.VMEM((EBLK, WS), jnp.float32),
        pltpu.VMEM((EBLK, WS), jnp.float32),
        pltpu.SemaphoreType.DMA,
        pltpu.SemaphoreType.DMA,
        pltpu.SemaphoreType.DMA,
        pltpu.SemaphoreType.DMA,
        pltpu.VMEM_SHARED((N, WS), jnp.float32),
    ],
)
def _sc_agg2(sa_hbm, sb_hbm, src_hbm, dst_hbm, out_hbm, sidx, didx,
             rows0, rows1, g0, g1, t0, t1, acc):
    cid = lax.axis_index("c")
    sid = lax.axis_index("s")

    @pl.when(cid == 0)
    def _():
        _init_rows(sa_hbm, acc, sid)

    @pl.when(cid != 0)
    def _():
        _init_rows(sb_hbm, acc, sid)

    plsc.subcore_barrier()

    def _run(s_hbm):
        def wrow(h, carry):
            w = sid * NC + h
            pltpu.sync_copy(src_hbm.at[w], sidx)
            pltpu.sync_copy(dst_hbm.at[w], didx)
            _pipe(s_hbm, acc, sidx, didx, rows0, rows1, g0, g1, t0, t1,
                  WBLK)
            return carry

        lax.fori_loop(0, NC, wrow, 0)

    @pl.when(cid == 0)
    def _():
        _run(sa_hbm)

    @pl.when(cid != 0)
    def _():
        _run(sb_hbm)

    plsc.subcore_barrier()
    _writeback_rows(acc, out_hbm, cid, sid)


_R = 1000


def _dinv(d):
    return lax.rsqrt(d[0, :, 0:1] + d[1, :, 0:1] + 1.0)


def _split(s):
    za = s[:, :WS]
    zb = jnp.concatenate(
        [s[:, WS:], jnp.zeros((s.shape[0], WS - WB), jnp.float32)], axis=1)
    return za, zb


def _tc0_body(x_ref, d_ref, o_ref):
    o_ref[...] = _dinv(d_ref[...]) * x_ref[...]


def _tc12_body(a_ref, d_ref, w1_ref, b1_ref, w2_ref, oa_ref, ob_ref):
    dinv = _dinv(d_ref[...])
    a = a_ref[0] + a_ref[1]
    h = jnp.maximum(
        dinv * jnp.dot(a, w1_ref[...], preferred_element_type=jnp.float32)
        + b1_ref[...], 0.0)
    s = dinv * jnp.dot(h, w2_ref[...], preferred_element_type=jnp.float32)
    oa_ref[...], ob_ref[...] = _split(s)


def _tc3_body(a_ref, d_ref, b_ref, wl_ref, bl_ref, cll_ref, o_ref):
    dinv = _dinv(d_ref[...])
    agg = jnp.concatenate([a_ref[0], a_ref[1, :, :WB]], axis=1)
    h = jnp.maximum(dinv * agg + b_ref[...], 0.0)
    emb = jnp.dot(h, wl_ref[...], preferred_element_type=jnp.float32)
    o_ref[:, 0:CLL] = cll_ref[...]
    o_ref[:, CLL:CLL + OUT_LL] = emb + bl_ref[...]


def _row_spec(w):
    return pl.BlockSpec((_R, w), lambda i: (i, 0))


def _pair_spec(w):
    return pl.BlockSpec((NC, _R, w), lambda i: (0, i, 0))


def _const_spec(shape):
    return pl.BlockSpec(shape, lambda i: tuple(0 for _ in shape))


_tc0 = pl.pallas_call(
    _tc0_body,
    grid=(N // _R,),
    in_specs=[_row_spec(F_IN), _pair_spec(DEGW)],
    out_specs=_row_spec(F_IN),
    out_shape=jax.ShapeDtypeStruct((N, F_IN), jnp.float32),
)

_tc12 = pl.pallas_call(
    _tc12_body,
    grid=(N // _R,),
    in_specs=[
        _pair_spec(F_IN),
        _pair_spec(DEGW),
        _const_spec((F_IN, HID)),
        _const_spec((1, HID)),
        _const_spec((HID, HID)),
    ],
    out_specs=[_row_spec(WS), _row_spec(WS)],
    out_shape=[jax.ShapeDtypeStruct((N, WS), jnp.float32),
               jax.ShapeDtypeStruct((N, WS), jnp.float32)],
)

_tc3 = pl.pallas_call(
    _tc3_body,
    grid=(N // _R,),
    in_specs=[
        _pair_spec(WS),
        _pair_spec(DEGW),
        _const_spec((1, HID)),
        _const_spec((HID, OUT_LL)),
        _const_spec((1, OUT_LL)),
        _row_spec(CLL),
    ],
    out_specs=_row_spec(CLL + OUT_LL),
    out_shape=jax.ShapeDtypeStruct((N, CLL + OUT_LL), jnp.float32),
)


def kernel(train_cll, train_drug, edge_index, W1, b1, W2, b2, Wl, bl):
    srcw = edge_index[0].reshape(NW, WBLK, EBLK)
    dstw = edge_index[1].reshape(NW, WBLK, EBLK)
    zeros_deg = jnp.zeros((N, DEGW), jnp.float32)
    ones_blk = jnp.ones((EBLK, DEGW), jnp.float32)
    zeros_f = jnp.zeros((N, F_IN), jnp.float32)

    deg = _sc_degree(dstw, zeros_deg, ones_blk)
    u1 = _tc0(train_drug, deg)
    a = _sc_agg_x(u1, zeros_f, srcw, dstw)
    s2a, s2b = _tc12(a, deg, W1, b1.reshape(1, HID), W2)
    g = _sc_agg2(s2a, s2b, srcw, dstw)
    out = _tc3(g, deg, b2.reshape(1, HID), Wl,
               bl.reshape(1, OUT_LL), train_cll)
    return out


# speedup vs baseline: 1.0181x; 1.0181x over previous
import functools

import jax
import jax.numpy as jnp
from jax import lax
from jax.experimental import pallas as pl
from jax.experimental.pallas import tpu as pltpu
from jax.experimental.pallas import tpu_sc as plsc

N = 10000
E = 320000
F_IN = 128
HID = 200
WS = 128
WB = HID - WS
OUT_LL = 100
CLL = 128

NC = 2
NS = 16
NW = NC * NS
EPW = E // NW
EBLK = 80
WBLK = EPW // EBLK
RPT = 624
TAIL = N - RPT * NS
DEGW = 8

_mesh = plsc.VectorSubcoreMesh(core_axis_name="c", subcore_axis_name="s")
_sc_params = pltpu.CompilerParams(use_tc_tiling_on_sc=False)


def _init_rows(src_hbm, acc, sid):
    r0 = sid * RPT
    pltpu.sync_copy(src_hbm.at[pl.ds(r0, RPT)], acc.at[pl.ds(r0, RPT)])

    @pl.when(sid == NS - 1)
    def _():
        pltpu.sync_copy(src_hbm.at[pl.ds(N - TAIL, TAIL)],
                        acc.at[pl.ds(N - TAIL, TAIL)])


def _writeback_rows(acc, out_hbm, cid, sid):
    r0 = sid * RPT
    pltpu.sync_copy(acc.at[pl.ds(r0, RPT)], out_hbm.at[cid, pl.ds(r0, RPT)])

    @pl.when(sid == NS - 1)
    def _():
        pltpu.sync_copy(acc.at[pl.ds(N - TAIL, TAIL)],
                        out_hbm.at[cid, pl.ds(N - TAIL, TAIL)])


def _pipe(s_hbm, acc, sblk, dblk, rows0, rows1, g0, g1, t0, t1, nblk):
    pltpu.async_copy(s_hbm.at[sblk(0)], rows0, g0)

    def body(k, carry):
        @pl.when(k % 2 == 0)
        def _():
            @pl.when(k + 1 < nblk)
            def _():
                @pl.when(k >= 1)
                def _():
                    pltpu.make_async_copy(rows1, acc.at[dblk(0)], t1).wait()

                pltpu.async_copy(s_hbm.at[sblk(k + 1)], rows1, g1)

            pltpu.make_async_copy(s_hbm.at[sblk(0)], rows0, g0).wait()
            pltpu.async_copy(rows0, acc.at[dblk(k)], t0, add=True)

        @pl.when(k % 2 == 1)
        def _():
            @pl.when(k + 1 < nblk)
            def _():
                pltpu.make_async_copy(rows0, acc.at[dblk(0)], t0).wait()
                pltpu.async_copy(s_hbm.at[sblk(k + 1)], rows0, g0)

            pltpu.make_async_copy(s_hbm.at[sblk(1)], rows1, g1).wait()
            pltpu.async_copy(rows1, acc.at[dblk(k)], t1, add=True)

        return carry

    lax.fori_loop(0, nblk, body, 0)
    pltpu.make_async_copy(rows1, acc.at[dblk(0)], t1).wait()
    pltpu.make_async_copy(rows0, acc.at[dblk(0)], t0).wait()


@functools.partial(
    pl.kernel,
    out_type=jax.ShapeDtypeStruct((NC, N, DEGW), jnp.float32),
    mesh=_mesh,
    compiler_params=_sc_params,
    scratch_types=[
        pltpu.VMEM((WBLK, EBLK), jnp.int32),
        pltpu.VMEM((EBLK, DEGW), jnp.float32),
        pltpu.SemaphoreType.DMA,
        pltpu.VMEM_SHARED((N, DEGW), jnp.float32),
    ],
)
def _sc_degree(dst_hbm, zeros_hbm, ones_hbm, out_hbm, didx, ones_v, dsem,
               acc):
    cid = lax.axis_index("c")
    sid = lax.axis_index("s")
    wid = sid * NC + cid
    _init_rows(zeros_hbm, acc, sid)
    pltpu.sync_copy(ones_hbm, ones_v)
    pltpu.sync_copy(dst_hbm.at[wid], didx)
    plsc.subcore_barrier()

    def body(j, carry):
        pltpu.async_copy(ones_v, acc.at[didx.at[j]], dsem, add=True)
        return carry

    lax.fori_loop(0, WBLK, body, 0)

    def drain(j, carry):
        pltpu.make_async_copy(ones_v, acc.at[didx.at[0]], dsem).wait()
        return carry

    lax.fori_loop(0, WBLK, drain, 0)
    plsc.subcore_barrier()
    _writeback_rows(acc, out_hbm, cid, sid)


@functools.partial(
    pl.kernel,
    out_type=jax.ShapeDtypeStruct((NC, N, F_IN), jnp.float32),
    mesh=_mesh,
    compiler_params=_sc_params,
    scratch_types=[
        pltpu.VMEM((EPW,), jnp.int32),
        pltpu.VMEM((WBLK, EBLK), jnp.int32),
        pltpu.VMEM((EBLK, F_IN), jnp.float32),
        pltpu.VMEM((EBLK, F_IN), jnp.float32),
        pltpu.SemaphoreType.DMA,
        pltpu.SemaphoreType.DMA,
        pltpu.SemaphoreType.DMA,
        pltpu.SemaphoreType.DMA,
        pltpu.VMEM_SHARED((N, F_IN), jnp.float32),
    ],
)
def _sc_agg_x(u_hbm, zeros_hbm, src_hbm, dst_hbm, out_hbm, sidx, didx,
              rows0, rows1, g0, g1, t0, t1, acc):
    cid = lax.axis_index("c")
    sid = lax.axis_index("s")
    wid = sid * NC + cid

    @pl.when(cid == 0)
    def _():
        _init_rows(u_hbm, acc, sid)

    @pl.when(cid != 0)
    def _():
        _init_rows(zeros_hbm, acc, sid)

    pltpu.sync_copy(src_hbm.at[pl.ds(wid * EPW, EPW)], sidx)
    pltpu.sync_copy(dst_hbm.at[wid], didx)
    plsc.subcore_barrier()
    _pipe(u_hbm, acc, lambda k: sidx.at[pl.ds(k * EBLK, EBLK)],
          lambda k: didx.at[k], rows0, rows1, g0, g1, t0, t1, WBLK)
    plsc.subcore_barrier()
    _writeback_rows(acc, out_hbm, cid, sid)


@functools.partial(
    pl.kernel,
    out_type=jax.ShapeDtypeStruct((NC, N, WS), jnp.float32),
    mesh=_mesh,
    compiler_params=_sc_params,
    scratch_types=[
        pltpu.VMEM((EPW,), jnp.int32),
        pltpu.VMEM((WBLK, EBLK), jnp.int32),
        pltpu.VMEM((EBLK, WS), jnp.float32),
        pltpu.VMEM((EBLK, WS), jnp.float32),
        pltpu.SemaphoreType.DMA,
        pltpu.SemaphoreType.DMA,
        pltpu.SemaphoreType.DMA,
        pltpu.SemaphoreType.DMA,
        pltpu.VMEM_SHARED((N, WS), jnp.float32),
    ],
)
def _sc_agg2(sa_hbm, sb_hbm, src_hbm, dst_hbm, out_hbm, sidx, didx,
             rows0, rows1, g0, g1, t0, t1, acc):
    cid = lax.axis_index("c")
    sid = lax.axis_index("s")

    @pl.when(cid == 0)
    def _():
        _init_rows(sa_hbm, acc, sid)

    @pl.when(cid != 0)
    def _():
        _init_rows(sb_hbm, acc, sid)

    plsc.subcore_barrier()

    def _run(s_hbm):
        def wrow(h, carry):
            w = sid * NC + h
            pltpu.sync_copy(src_hbm.at[pl.ds(w * EPW, EPW)], sidx)
            pltpu.sync_copy(dst_hbm.at[w], didx)
            _pipe(s_hbm, acc, lambda k: sidx.at[pl.ds(k * EBLK, EBLK)],
                  lambda k: didx.at[k], rows0, rows1, g0, g1, t0, t1,
                  WBLK)
            return carry

        lax.fori_loop(0, NC, wrow, 0)

    @pl.when(cid == 0)
    def _():
        _run(sa_hbm)

    @pl.when(cid != 0)
    def _():
        _run(sb_hbm)

    plsc.subcore_barrier()
    _writeback_rows(acc, out_hbm, cid, sid)


_R = 1000


def _dinv(d):
    return lax.rsqrt(d[0, :, 0:1] + d[1, :, 0:1] + 1.0)


def _split(s):
    za = s[:, :WS]
    zb = jnp.concatenate(
        [s[:, WS:], jnp.zeros((s.shape[0], WS - WB), jnp.float32)], axis=1)
    return za, zb


def _tc0_body(x_ref, d_ref, o_ref):
    o_ref[...] = _dinv(d_ref[...]) * x_ref[...]


def _tc12_body(a_ref, d_ref, w1_ref, b1_ref, w2_ref, oa_ref, ob_ref):
    dinv = _dinv(d_ref[...])
    a = a_ref[0] + a_ref[1]
    h = jnp.maximum(
        dinv * jnp.dot(a, w1_ref[...], preferred_element_type=jnp.float32)
        + b1_ref[...], 0.0)
    s = dinv * jnp.dot(h, w2_ref[...], preferred_element_type=jnp.float32)
    oa_ref[...], ob_ref[...] = _split(s)


def _tc3_body(a_ref, d_ref, b_ref, wl_ref, bl_ref, cll_ref, o_ref):
    dinv = _dinv(d_ref[...])
    agg = jnp.concatenate([a_ref[0], a_ref[1, :, :WB]], axis=1)
    h = jnp.maximum(dinv * agg + b_ref[...], 0.0)
    emb = jnp.dot(h, wl_ref[...], preferred_element_type=jnp.float32)
    o_ref[:, 0:CLL] = cll_ref[...]
    o_ref[:, CLL:CLL + OUT_LL] = emb + bl_ref[...]


def _row_spec(w):
    return pl.BlockSpec((_R, w), lambda i: (i, 0))


def _pair_spec(w):
    return pl.BlockSpec((NC, _R, w), lambda i: (0, i, 0))


def _const_spec(shape):
    return pl.BlockSpec(shape, lambda i: tuple(0 for _ in shape))


_tc0 = pl.pallas_call(
    _tc0_body,
    grid=(N // _R,),
    in_specs=[_row_spec(F_IN), _pair_spec(DEGW)],
    out_specs=_row_spec(F_IN),
    out_shape=jax.ShapeDtypeStruct((N, F_IN), jnp.float32),
)

_tc12 = pl.pallas_call(
    _tc12_body,
    grid=(N // _R,),
    in_specs=[
        _pair_spec(F_IN),
        _pair_spec(DEGW),
        _const_spec((F_IN, HID)),
        _const_spec((1, HID)),
        _const_spec((HID, HID)),
    ],
    out_specs=[_row_spec(WS), _row_spec(WS)],
    out_shape=[jax.ShapeDtypeStruct((N, WS), jnp.float32),
               jax.ShapeDtypeStruct((N, WS), jnp.float32)],
)

_tc3 = pl.pallas_call(
    _tc3_body,
    grid=(N // _R,),
    in_specs=[
        _pair_spec(WS),
        _pair_spec(DEGW),
        _const_spec((1, HID)),
        _const_spec((HID, OUT_LL)),
        _const_spec((1, OUT_LL)),
        _row_spec(CLL),
    ],
    out_specs=_row_spec(CLL + OUT_LL),
    out_shape=jax.ShapeDtypeStruct((N, CLL + OUT_LL), jnp.float32),
)


def kernel(train_cll, train_drug, edge_index, W1, b1, W2, b2, Wl, bl):
    src1 = edge_index[0]
    dstw = edge_index[1].reshape(NW, WBLK, EBLK)
    zeros_deg = jnp.zeros((N, DEGW), jnp.float32)
    ones_blk = jnp.ones((EBLK, DEGW), jnp.float32)
    zeros_f = jnp.zeros((N, F_IN), jnp.float32)

    deg = _sc_degree(dstw, zeros_deg, ones_blk)
    u1 = _tc0(train_drug, deg)
    a = _sc_agg_x(u1, zeros_f, src1, dstw)
    s2a, s2b = _tc12(a, deg, W1, b1.reshape(1, HID), W2)
    g = _sc_agg2(s2a, s2b, src1, dstw)
    out = _tc3(g, deg, b2.reshape(1, HID), Wl,
               bl.reshape(1, OUT_LL), train_cll)
    return out
